# baseline (speedup 1.0000x reference)
import functools

import jax
import jax.numpy as jnp
from jax import lax
from jax.experimental import pallas as pl
from jax.experimental.pallas import tpu as pltpu
from jax.experimental.pallas import tpu_sc as plsc

B = 16384
D = 64
K = 5

NC = 2
NS = 16
NW = NC * NS
BPW = B // NW
GCH = 128
NCH = BPW // GCH
NGRP = BPW // 16


def _sc_kernel(uidx_hbm, iidx_hbm, uemb_hbm, iemb_hbm, w_hbm, b_hbm,
               out_hbm, idx_u, idx_i, u_rows, v_rows, w_v, b_v, out_v, sem):
    wid = lax.axis_index("s") * NC + lax.axis_index("c")
    base = wid * BPW

    pltpu.sync_copy(uidx_hbm.at[wid], idx_u)
    pltpu.sync_copy(iidx_hbm.at[wid], idx_i)
    pltpu.sync_copy(w_hbm, w_v)
    pltpu.sync_copy(b_hbm, b_v)

    copies = []
    for j in range(NCH):
        copies.append(pltpu.async_copy(
            uemb_hbm.at[idx_u.at[j]], u_rows.at[pl.ds(j * GCH, GCH)], sem))
        copies.append(pltpu.async_copy(
            iemb_hbm.at[idx_i.at[j]], v_rows.at[pl.ds(j * GCH, GCH)], sem))
    for c in copies:
        c.wait()

    lane = lax.iota(jnp.int32, 16)
    bvecs = tuple(b_v[k] for k in range(K))
    kvecs = tuple(jnp.full((16,), k, jnp.int32) for k in range(K))

    def group_body(g, carry):
        rows = g * 16 + lane
        accs = bvecs

        def d_body(d, accs):
            dvec = jnp.full((16,), d, dtype=jnp.int32)
            u_d = plsc.load_gather(u_rows, [rows, dvec])
            v_d = plsc.load_gather(v_rows, [rows, dvec])
            m = u_d * v_d
            wk = tuple(w_v[d * K + k] for k in range(K))
            return tuple(accs[k] + m * wk[k] for k in range(K))

        accs = lax.fori_loop(0, D, d_body, accs)
        for k in range(K):
            p = 1.0 / (1.0 + jnp.exp(-accs[k]))
            plsc.store_scatter(out_v, [rows, kvecs[k]], p)
        return carry

    lax.fori_loop(0, NGRP, group_body, 0)
    pltpu.sync_copy(out_v, out_hbm.at[pl.ds(base, BPW)])


@jax.jit
def _bpr(uidx, iidx, user_emb, item_emb, w_pad, b_pad):
    mesh = plsc.VectorSubcoreMesh(core_axis_name="c", subcore_axis_name="s")
    run = functools.partial(
        pl.kernel,
        out_type=jax.ShapeDtypeStruct((B, K), jnp.float32),
        mesh=mesh,
        compiler_params=pltpu.CompilerParams(
            needs_layout_passes=False, use_tc_tiling_on_sc=False),
        scratch_types=[
            pltpu.VMEM((NCH, GCH), jnp.int32),
            pltpu.VMEM((NCH, GCH), jnp.int32),
            pltpu.VMEM((BPW, D), jnp.float32),
            pltpu.VMEM((BPW, D), jnp.float32),
            pltpu.VMEM((D * K, 16), jnp.float32),
            pltpu.VMEM((8, 16), jnp.float32),
            pltpu.VMEM((BPW, K), jnp.float32),
            pltpu.SemaphoreType.DMA,
        ],
    )(_sc_kernel)
    return run(uidx, iidx, user_emb, item_emb, w_pad, b_pad)


def kernel(user_input, item_input, user_emb, item_emb, W, b):
    uidx = user_input.astype(jnp.int32).reshape(NW, NCH, GCH)
    iidx = item_input.astype(jnp.int32).reshape(NW, NCH, GCH)
    w_bc = jnp.broadcast_to(W.T.reshape(D, K, 1), (D, K, 16)).reshape(D * K, 16)
    b_bc = jnp.zeros((8, 16), jnp.float32).at[:K].set(
        jnp.broadcast_to(b.reshape(K, 1), (K, 16)))
    return _bpr(uidx, iidx, user_emb, item_emb, w_bc, b_bc)
